# Initial kernel scaffold; baseline (speedup 1.0000x reference)
import jax
import jax.numpy as jnp
from jax.experimental import pallas as pl


def kernel(z, pos, edge_index, emb, mlp_w1, mlp_b1, mlp_w2, mlp_b2, conv_w1, conv_w2, conv_b2, lin_w, lin_b, l1_w, l1_b, l2_w, l2_b):
    raise NotImplementedError("write your pallas kernel here")


# speedup vs baseline: 1.0883x; 1.0883x over previous
import functools
import math

import jax
import jax.numpy as jnp
from jax import lax
from jax.experimental import pallas as pl
from jax.experimental.pallas import tpu as pltpu
from jax.experimental.pallas import tpu_sc as plsc

N = 50000
E = 800000
H = 64
NG = 50
NI = 3
CUT = 10.0
LOG2 = math.log(2.0)

NSC = 2
NT = 16
CH = 128

NCK = 391
EPT = NCK * CH
EP = NT * EPT

ZPT = 3200
NH = NT * NSC * ZPT // 2
NPT = N // NT
NZC = 25
NZR = NPT // NZC

BE = 2048
BN = 2000

@functools.cache
def _get_mesh():
    return plsc.VectorSubcoreMesh(core_axis_name="c", subcore_axis_name="s",
                                  num_cores=NSC, num_subcores=NT)


def _ssp(x):
    return jax.nn.softplus(x) - LOG2


@functools.cache
def _get_embgeom():
    @functools.partial(
        pl.kernel,
        mesh=_get_mesh(),
        compiler_params=pltpu.CompilerParams(use_tc_tiling_on_sc=False),
        out_type=[
            jax.ShapeDtypeStruct((NH, H), jnp.float32),
            jax.ShapeDtypeStruct((EP, 16), jnp.float32),
            jax.ShapeDtypeStruct((EP, 16), jnp.float32),
        ],
        scratch_types=[
            pltpu.VMEM((CH,), jnp.int32),
            pltpu.VMEM((CH, H), jnp.float32),
            pltpu.VMEM((CH,), jnp.int32),
            pltpu.VMEM((CH,), jnp.int32),
            pltpu.VMEM((CH, 16), jnp.float32),
            pltpu.VMEM((CH, 16), jnp.float32),
            pltpu.SemaphoreType.DMA,
        ],
    )
    def _embgeom(z_h, emb_h, pos_h, row_h, col_h, h_out, pr_out, pc_out,
                 zidx, hbuf, ridx, cidx, rpos, cpos, sem):
        c = lax.axis_index("c")
        s = lax.axis_index("s")

        @pl.when(c == 0)
        def _():
            base = s * ZPT

            def body(k, carry):
                off = base + k * CH
                pltpu.sync_copy(z_h.at[pl.ds(off, CH)], zidx)
                pltpu.async_copy(emb_h.at[zidx], hbuf, sem).wait()
                pltpu.sync_copy(hbuf, h_out.at[pl.ds(off, CH)])
                return carry

            lax.fori_loop(0, ZPT // CH, body, 0)

        @pl.when(c == 1)
        def _():
            base = s * EPT

            def body(k, carry):
                off = base + k * CH
                pltpu.sync_copy(row_h.at[pl.ds(off, CH)], ridx)
                pltpu.sync_copy(col_h.at[pl.ds(off, CH)], cidx)
                pltpu.async_copy(pos_h.at[ridx], rpos, sem).wait()
                pltpu.async_copy(pos_h.at[cidx], cpos, sem).wait()
                pltpu.sync_copy(rpos, pr_out.at[pl.ds(off, CH)])
                pltpu.sync_copy(cpos, pc_out.at[pl.ds(off, CH)])
                return carry

            lax.fori_loop(0, NCK, body, 0)

    return _embgeom


@functools.cache
def _make_gms(layer):
    @functools.partial(
        pl.kernel,
        mesh=_get_mesh(),
        compiler_params=pltpu.CompilerParams(use_tc_tiling_on_sc=False),
        out_type=jax.ShapeDtypeStruct((2, N, 32), jnp.float32),
        scratch_types=[
            pltpu.VMEM((CH,), jnp.int32),
            pltpu.VMEM((CH,), jnp.int32),
            pltpu.VMEM((CH, 32), jnp.float32),
            pltpu.VMEM((CH, 32), jnp.float32),
            pltpu.VMEM_SHARED((N, 32), jnp.float32),
            pltpu.SemaphoreType.DMA,
        ],
    )
    def _gms(xh_h, wf_h, row_h, col_h, zeros_h, out_h,
             ridx, cidx, gbuf, wbuf, agg, sem):
        c = lax.axis_index("c")
        s = lax.axis_index("s")

        def zb(k, carry):
            off = s * NPT + k * NZR
            pltpu.sync_copy(zeros_h, agg.at[pl.ds(off, NZR)])
            return carry

        lax.fori_loop(0, NZC, zb, 0)
        plsc.subcore_barrier()

        base = s * EPT
        wf_l = wf_h.at[layer]

        def body(k, carry):
            off = base + k * CH
            pltpu.sync_copy(row_h.at[pl.ds(off, CH)], ridx)
            pltpu.sync_copy(col_h.at[pl.ds(off, CH)], cidx)
            pltpu.async_copy(xh_h.at[c].at[ridx], gbuf, sem).wait()
            pltpu.sync_copy(wf_l.at[c].at[pl.ds(off, CH)], wbuf)

            def mul(j, cc):
                gbuf[j, pl.ds(0, 16)] = gbuf[j, pl.ds(0, 16)] * wbuf[j, pl.ds(0, 16)]
                gbuf[j, pl.ds(16, 16)] = gbuf[j, pl.ds(16, 16)] * wbuf[j, pl.ds(16, 16)]
                return cc

            lax.fori_loop(0, CH, mul, 0)
            pltpu.sync_copy(gbuf, agg.at[cidx], add=True)
            return carry

        lax.fori_loop(0, NCK, body, 0)
        plsc.subcore_barrier()
        pltpu.sync_copy(agg.at[pl.ds(s * NPT, NPT)],
                        out_h.at[c].at[pl.ds(s * NPT, NPT)])

    return _gms


def _wf_body(pr_ref, pc_ref, w1_ref, b1_ref, w2w_ref, b2_ref, out_ref):
    e = pl.program_id(1)
    delta = CUT / (NG - 1)
    coeff = -0.5 / (delta * delta)
    d = pr_ref[...] - pc_ref[...]
    w2v = jnp.sum(d * d, axis=1, keepdims=True)
    w = jnp.sqrt(w2v + 1e-12)
    off = lax.broadcasted_iota(jnp.int32, (1, NG), 1).astype(jnp.float32) * delta
    ea = jnp.exp(coeff * (w - off) ** 2)
    t = jnp.dot(ea, w1_ref[0], preferred_element_type=jnp.float32) + b1_ref[0]
    t = _ssp(t)
    wf = jnp.dot(t, w2w_ref[0], preferred_element_type=jnp.float32) + b2_ref[0]
    cfac = 0.5 * (jnp.cos(w * (math.pi / CUT)) + 1.0)
    gid = e * BE + lax.broadcasted_iota(jnp.int32, (BE, 1), 0)
    cfac = jnp.where(gid < E, cfac, 0.0)
    wf = wf * cfac
    out_ref[0, 0] = wf[:, :32]
    out_ref[0, 1] = wf[:, 32:]


def _wf_all():
    return pl.pallas_call(
        _wf_body,
        grid=(NI, EP // BE),
        in_specs=[
            pl.BlockSpec((BE, 16), lambda l, e: (e, 0)),
            pl.BlockSpec((BE, 16), lambda l, e: (e, 0)),
            pl.BlockSpec((1, NG, H), lambda l, e: (l, 0, 0)),
            pl.BlockSpec((1, 1, H), lambda l, e: (l, 0, 0)),
            pl.BlockSpec((1, H, H), lambda l, e: (l, 0, 0)),
            pl.BlockSpec((1, 1, H), lambda l, e: (l, 0, 0)),
        ],
        out_specs=pl.BlockSpec((1, 2, BE, 32), lambda l, e: (l, 0, e, 0)),
        out_shape=jax.ShapeDtypeStruct((NI, 2, EP, 32), jnp.float32),
    )


def _xsplit_body(h_ref, w_ref, out_ref):
    x = jnp.dot(h_ref[...], w_ref[...], preferred_element_type=jnp.float32)
    out_ref[0] = x[:, :32]
    out_ref[1] = x[:, 32:]


def _xsplit(h, w):
    return pl.pallas_call(
        _xsplit_body,
        grid=(N // BN,),
        in_specs=[
            pl.BlockSpec((BN, H), lambda i: (i, 0)),
            pl.BlockSpec((H, H), lambda i: (0, 0)),
        ],
        out_specs=pl.BlockSpec((2, BN, 32), lambda i: (0, i, 0)),
        out_shape=jax.ShapeDtypeStruct((2, N, 32), jnp.float32),
    )(h, w)


def _epi_body(final, agg_ref, h_ref, cw2_ref, cb2_ref, lw_ref, lb_ref,
              wa_ref, wb_ref, hn_ref, xh_ref):
    agg = jnp.concatenate([agg_ref[0], agg_ref[1]], axis=1)
    x = jnp.dot(agg, cw2_ref[...], preferred_element_type=jnp.float32) + cb2_ref[...]
    x = _ssp(x)
    x = jnp.dot(x, lw_ref[...], preferred_element_type=jnp.float32) + lb_ref[...]
    hn = h_ref[...] + x
    if final:
        t = jnp.dot(hn, wa_ref[...], preferred_element_type=jnp.float32) + wb_ref[...]
        t = _ssp(t)
        hn_ref[...] = t
        xh_ref[0] = hn[:, :32]
        xh_ref[1] = hn[:, 32:]
    else:
        hn_ref[...] = hn
        xn = jnp.dot(hn, wa_ref[...], preferred_element_type=jnp.float32)
        xh_ref[0] = xn[:, :32]
        xh_ref[1] = xn[:, 32:]


def _epilogue(final, agg, h, cw2, cb2, lw, lb, wa, wb):
    return pl.pallas_call(
        functools.partial(_epi_body, final),
        grid=(N // BN,),
        in_specs=[
            pl.BlockSpec((2, BN, 32), lambda i: (0, i, 0)),
            pl.BlockSpec((BN, H), lambda i: (i, 0)),
            pl.BlockSpec((H, H), lambda i: (0, 0)),
            pl.BlockSpec((1, H), lambda i: (0, 0)),
            pl.BlockSpec((H, H), lambda i: (0, 0)),
            pl.BlockSpec((1, H), lambda i: (0, 0)),
            pl.BlockSpec((H, H), lambda i: (0, 0)),
            pl.BlockSpec((1, H), lambda i: (0, 0)),
        ],
        out_specs=[
            pl.BlockSpec((BN, H), lambda i: (i, 0)),
            pl.BlockSpec((2, BN, 32), lambda i: (0, i, 0)),
        ],
        out_shape=[
            jax.ShapeDtypeStruct((N, H), jnp.float32),
            jax.ShapeDtypeStruct((2, N, 32), jnp.float32),
        ],
    )(agg, h, cw2, cb2, lw, lb, wa, wb)


def _final_matmul_body(h_ref, w_ref, b_ref, out_ref):
    out_ref[...] = (
        jnp.dot(h_ref[...], w_ref[...], preferred_element_type=jnp.float32)
        + b_ref[...]
    )


def _final_matmul(h, w, b):
    return pl.pallas_call(
        _final_matmul_body,
        grid=(N // BN,),
        in_specs=[
            pl.BlockSpec((BN, H), lambda i: (i, 0)),
            pl.BlockSpec((H, H), lambda i: (0, 0)),
            pl.BlockSpec((1, H), lambda i: (0, 0)),
        ],
        out_specs=pl.BlockSpec((BN, H), lambda i: (i, 0)),
        out_shape=jax.ShapeDtypeStruct((N, H), jnp.float32),
    )(h, w, b)


def kernel(z, pos, edge_index, emb, mlp_w1, mlp_b1, mlp_w2, mlp_b2,
           conv_w1, conv_w2, conv_b2, lin_w, lin_b, l1_w, l1_b, l2_w, l2_b):
    z32 = jnp.pad(z.astype(jnp.int32), (0, NH - N))
    row = jnp.pad(edge_index[0].astype(jnp.int32), (0, EP - E))
    col = jnp.pad(edge_index[1].astype(jnp.int32), (0, EP - E))
    pos16 = jnp.pad(pos.astype(jnp.float32), ((0, 0), (0, 13)))
    zeros_blk = jnp.zeros((NZR, 32), jnp.float32)

    h_big, pr, pc = _get_embgeom()(z32, emb, pos16, row, col)
    wf = _wf_all()(pr, pc, mlp_w1, mlp_b1.reshape(NI, 1, H), mlp_w2,
                   mlp_b2.reshape(NI, 1, H))

    h = h_big
    xh = _xsplit(h_big, conv_w1[0])
    for i in range(NI):
        agg = _make_gms(i)(xh, wf, row, col, zeros_blk)
        final = i == NI - 1
        wa = l1_w if final else conv_w1[i + 1]
        wb = l1_b.reshape(1, H) if final else jnp.zeros((1, H), jnp.float32)
        h, xh = _epilogue(final, agg, h, conv_w2[i], conv_b2[i].reshape(1, H),
                          lin_w[i], lin_b[i].reshape(1, H), wa, wb)
    out = _final_matmul(h, l2_w, l2_b.reshape(1, H))
    return out
